# baseline (speedup 1.0000x reference)
import functools

import jax
import jax.numpy as jnp
from jax import lax
from jax.experimental import pallas as pl
from jax.experimental.pallas import tpu as pltpu
from jax.experimental.pallas import tpu_sc as plsc

_BM = 2000
_FRACS = (13, 7)
_K = len(_FRACS)
_G = 128
_GPC = 8
_NSUB = 16
_NW = 2 * _NSUB
_ACC = 10240


def _mm_body(x_ref, w_ref, b_ref, o_ref):
    y = jnp.dot(x_ref[...], w_ref[...], preferred_element_type=jnp.float32)
    o_ref[...] = jnp.maximum(y + b_ref[...], 0.0)


def _matmul_relu(x, row0, rows, rows_padded, w, b):
    d_in = x.shape[1]
    d_out = w.shape[1]
    off = row0 // _BM
    return pl.pallas_call(
        _mm_body,
        grid=(rows // _BM,),
        in_specs=[
            pl.BlockSpec((_BM, d_in), lambda k: (k + off, 0)),
            pl.BlockSpec((d_in, d_out), lambda k: (0, 0)),
            pl.BlockSpec((1, d_out), lambda k: (0, 0)),
        ],
        out_specs=pl.BlockSpec((_BM, d_out), lambda k: (k, 0)),
        out_shape=jax.ShapeDtypeStruct((rows_padded, d_out), jnp.float32),
    )(x, w, b.reshape(1, d_out))


def _segment_partials_sc(p, idx2d):
    d = p.shape[1]
    n_rows = idx2d.shape[0]
    rpw = n_rows // _NW
    n_groups = rpw // _GPC
    mpt = _ACC // _NSUB
    assert rpw * _NW == n_rows and rpw % _GPC == 0
    mesh = plsc.VectorSubcoreMesh(core_axis_name="c", subcore_axis_name="s")

    @functools.partial(
        pl.kernel,
        mesh=mesh,
        out_type=(
            jax.ShapeDtypeStruct((2, _ACC, d), jnp.float32),
            jax.ShapeDtypeStruct((2, 1, _ACC), jnp.float32),
        ),
        scratch_types=[
            pltpu.VMEM((_G, d), jnp.float32),
            pltpu.VMEM((_G, d), jnp.float32),
            pltpu.VMEM((n_groups, _GPC, _G), jnp.int32),
            pltpu.VMEM((_G,), jnp.float32),
            pltpu.VMEM((mpt,), jnp.float32),
            pltpu.SemaphoreType.DMA,
            pltpu.SemaphoreType.DMA,
            pltpu.SemaphoreType.DMA,
            pltpu.VMEM_SHARED((_ACC, d), jnp.float32),
            pltpu.VMEM_SHARED((_ACC,), jnp.float32),
        ],
    )
    def seg_part(p_hbm, idx_hbm, sums_out, cnts_out, buf0, buf1, idx_v,
                 ones_v, zero_v, sem0, sem1, csem, sums_sh, cnts_sh):
        c = lax.axis_index("c")
        s = lax.axis_index("s")
        w = c * _NSUB + s
        bufs = (buf0, buf1)
        sems = (sem0, sem1)

        zeros16 = jnp.zeros((16,), jnp.float32)
        ones16 = jnp.ones((16,), jnp.float32)

        def zero_body(m, carry):
            for j in range(d // 16):
                buf0[m, pl.ds(16 * j, 16)] = zeros16
            return carry

        lax.fori_loop(0, _G, zero_body, 0)
        for k in range(_G // 16):
            ones_v[pl.ds(16 * k, 16)] = ones16
        for k in range(mpt // 16):
            zero_v[pl.ds(16 * k, 16)] = zeros16

        z0 = s * mpt
        for k in range(mpt // _G):
            pltpu.sync_copy(buf0, sums_sh.at[pl.ds(z0 + k * _G, _G)])
        pltpu.sync_copy(zero_v, cnts_sh.at[pl.ds(z0, mpt)])
        plsc.subcore_barrier()

        row_base = w * rpw

        def src(pc):
            return p_hbm.at[pl.ds((row_base + pc) * _G, _G)]

        for pc in range(2):
            pltpu.async_copy(src(pc), bufs[pc], sems[pc])

        def group_body(t, carry):
            base = t * _GPC
            pltpu.sync_copy(idx_hbm.at[pl.ds(row_base + base, _GPC)],
                            idx_v.at[t])
            for j in range(_GPC):
                b = j % 2
                pc = base + j
                pltpu.make_async_copy(src(pc), bufs[b], sems[b]).wait()
                pltpu.sync_copy(bufs[b], sums_sh.at[idx_v.at[t].at[j]],
                                add=True)
                pltpu.async_copy(ones_v, cnts_sh.at[idx_v.at[t].at[j]],
                                 csem, add=True)

                @pl.when(pc + 2 < rpw)
                def _next():
                    pltpu.async_copy(src(pc + 2), bufs[b], sems[b])
            return carry

        lax.fori_loop(0, n_groups, group_body, 0)

        for t in range(n_groups):
            for j in range(_GPC):
                pltpu.make_async_copy(
                    ones_v, cnts_sh.at[idx_v.at[t].at[j]], csem).wait()
        plsc.subcore_barrier()

        pltpu.sync_copy(sums_sh.at[pl.ds(z0, mpt)],
                        sums_out.at[c].at[pl.ds(z0, mpt)])

        @pl.when(s == 0)
        def _dump_counts():
            pltpu.sync_copy(cnts_sh, cnts_out.at[c].at[0])

    return seg_part(p, idx2d)


_CM = 1024


def _combine_body(*refs):
    s_refs = refs[:_K]
    c_refs = refs[_K:2 * _K]
    o_ref = refs[2 * _K]
    cs = sum(c_ref[0] + c_ref[1] for c_ref in c_refs)
    ss = sum(s_ref[0] + s_ref[1] for s_ref in s_refs)
    r = 1.0 / jnp.maximum(cs, 1.0)
    o_ref[...] = ss * r[:, None]


def _combine(sums_list, cnts_list, n_mols, d):
    grid = (-(-n_mols // _CM),)
    return pl.pallas_call(
        _combine_body,
        grid=grid,
        in_specs=(
            [pl.BlockSpec((2, _CM, d), lambda k: (0, k, 0))] * _K
            + [pl.BlockSpec((2, _CM), lambda k: (0, k))] * _K
        ),
        out_specs=pl.BlockSpec((_CM, d), lambda k: (k, 0)),
        out_shape=jax.ShapeDtypeStruct((n_mols, d), jnp.float32),
    )(*sums_list, *cnts_list)


def kernel(vi, atom_mol_batch, N, W, b):
    n_mols = N.shape[0]
    n_atoms = vi.shape[0] - n_mols
    d = W.shape[1]
    chunk_sizes = [n_atoms * f // sum(_FRACS) for f in _FRACS]
    chunk_sizes[-1] += n_atoms - sum(chunk_sizes)
    sums_list, cnts_list = [], []
    row0 = 0
    for cr in chunk_sizes:
        rpw = _GPC * (-(-cr // (_G * _GPC * _NW)))
        n_rows = rpw * _NW
        p_k = _matmul_relu(vi, row0, cr, n_rows * _G, W, b)
        idx_k = lax.dynamic_slice_in_dim(atom_mol_batch, row0, cr)
        idx_pad = jnp.full((n_rows * _G - cr,), n_mols, dtype=jnp.int32)
        idx2d = jnp.concatenate([idx_k, idx_pad]).reshape(n_rows, _G)
        s_k, c_k = _segment_partials_sc(p_k, idx2d)
        sums_list.append(s_k)
        cnts_list.append(c_k.reshape(2, _ACC))
        row0 += cr
    return _combine(sums_list, cnts_list, n_mols, d)


# speedup vs baseline: 1.1710x; 1.1710x over previous
import functools

import jax
import jax.numpy as jnp
from jax import lax
from jax.experimental import pallas as pl
from jax.experimental.pallas import tpu as pltpu
from jax.experimental.pallas import tpu_sc as plsc

_BM = 4000
_FRACS = (3, 2)
_K = len(_FRACS)
_G = 128
_GPC = 8
_NSUB = 16
_NW = 2 * _NSUB
_ACC = 10240


def _mm_body(x_ref, w_ref, b_ref, o_ref):
    y = jnp.dot(x_ref[...], w_ref[...], preferred_element_type=jnp.float32)
    o_ref[...] = jnp.maximum(y + b_ref[...], 0.0)


def _matmul_relu(x, row0, rows, rows_padded, w, b):
    d_in = x.shape[1]
    d_out = w.shape[1]
    off = row0 // _BM
    return pl.pallas_call(
        _mm_body,
        grid=(rows // _BM,),
        in_specs=[
            pl.BlockSpec((_BM, d_in), lambda k: (k + off, 0)),
            pl.BlockSpec((d_in, d_out), lambda k: (0, 0)),
            pl.BlockSpec((1, d_out), lambda k: (0, 0)),
        ],
        out_specs=pl.BlockSpec((_BM, d_out), lambda k: (k, 0)),
        out_shape=jax.ShapeDtypeStruct((rows_padded, d_out), jnp.float32),
    )(x, w, b.reshape(1, d_out))


def _segment_partials_sc(p, idx2d):
    d = p.shape[1]
    n_rows = idx2d.shape[0]
    rpw = n_rows // _NW
    n_groups = rpw // _GPC
    mpt = _ACC // _NSUB
    assert rpw * _NW == n_rows and rpw % _GPC == 0
    mesh = plsc.VectorSubcoreMesh(core_axis_name="c", subcore_axis_name="s")

    @functools.partial(
        pl.kernel,
        mesh=mesh,
        out_type=(
            jax.ShapeDtypeStruct((2, _ACC, d), jnp.float32),
            jax.ShapeDtypeStruct((2, 1, _ACC), jnp.float32),
        ),
        scratch_types=[
            pltpu.VMEM((_G, d), jnp.float32),
            pltpu.VMEM((_G, d), jnp.float32),
            pltpu.VMEM((n_groups, _GPC, _G), jnp.int32),
            pltpu.VMEM((_G,), jnp.float32),
            pltpu.VMEM((mpt,), jnp.float32),
            pltpu.SemaphoreType.DMA,
            pltpu.SemaphoreType.DMA,
            pltpu.SemaphoreType.DMA,
            pltpu.VMEM_SHARED((_ACC, d), jnp.float32),
            pltpu.VMEM_SHARED((_ACC,), jnp.float32),
        ],
    )
    def seg_part(p_hbm, idx_hbm, sums_out, cnts_out, buf0, buf1, idx_v,
                 ones_v, zero_v, sem0, sem1, csem, sums_sh, cnts_sh):
        c = lax.axis_index("c")
        s = lax.axis_index("s")
        w = c * _NSUB + s
        bufs = (buf0, buf1)
        sems = (sem0, sem1)

        zeros16 = jnp.zeros((16,), jnp.float32)
        ones16 = jnp.ones((16,), jnp.float32)

        def zero_body(m, carry):
            for j in range(d // 16):
                buf0[m, pl.ds(16 * j, 16)] = zeros16
            return carry

        lax.fori_loop(0, _G, zero_body, 0)
        for k in range(_G // 16):
            ones_v[pl.ds(16 * k, 16)] = ones16
        for k in range(mpt // 16):
            zero_v[pl.ds(16 * k, 16)] = zeros16

        z0 = s * mpt
        for k in range(mpt // _G):
            pltpu.sync_copy(buf0, sums_sh.at[pl.ds(z0 + k * _G, _G)])
        pltpu.sync_copy(zero_v, cnts_sh.at[pl.ds(z0, mpt)])
        plsc.subcore_barrier()

        row_base = w * rpw

        def src(pc):
            return p_hbm.at[pl.ds((row_base + pc) * _G, _G)]

        for pc in range(2):
            pltpu.async_copy(src(pc), bufs[pc], sems[pc])

        def group_body(t, carry):
            base = t * _GPC
            pltpu.sync_copy(idx_hbm.at[pl.ds(row_base + base, _GPC)],
                            idx_v.at[t])
            for j in range(_GPC):
                b = j % 2
                pc = base + j
                pltpu.make_async_copy(src(pc), bufs[b], sems[b]).wait()
                pltpu.sync_copy(bufs[b], sums_sh.at[idx_v.at[t].at[j]],
                                add=True)
                pltpu.async_copy(ones_v, cnts_sh.at[idx_v.at[t].at[j]],
                                 csem, add=True)

                @pl.when(pc + 2 < rpw)
                def _next():
                    pltpu.async_copy(src(pc + 2), bufs[b], sems[b])
            return carry

        lax.fori_loop(0, n_groups, group_body, 0)

        for t in range(n_groups):
            for j in range(_GPC):
                pltpu.make_async_copy(
                    ones_v, cnts_sh.at[idx_v.at[t].at[j]], csem).wait()
        plsc.subcore_barrier()

        pltpu.sync_copy(sums_sh.at[pl.ds(z0, mpt)],
                        sums_out.at[c].at[pl.ds(z0, mpt)])

        @pl.when(s == 0)
        def _dump_counts():
            pltpu.sync_copy(cnts_sh, cnts_out.at[c].at[0])

    return seg_part(p, idx2d)


_CM = 1024


def _combine_body(*refs):
    s_refs = refs[:_K]
    c_refs = refs[_K:2 * _K]
    o_ref = refs[2 * _K]
    cs = sum(c_ref[0] + c_ref[1] for c_ref in c_refs)
    ss = sum(s_ref[0] + s_ref[1] for s_ref in s_refs)
    r = 1.0 / jnp.maximum(cs, 1.0)
    o_ref[...] = ss * r[:, None]


def _combine(sums_list, cnts_list, n_mols, d):
    grid = (-(-n_mols // _CM),)
    return pl.pallas_call(
        _combine_body,
        grid=grid,
        in_specs=(
            [pl.BlockSpec((2, _CM, d), lambda k: (0, k, 0))] * _K
            + [pl.BlockSpec((2, _CM), lambda k: (0, k))] * _K
        ),
        out_specs=pl.BlockSpec((_CM, d), lambda k: (k, 0)),
        out_shape=jax.ShapeDtypeStruct((n_mols, d), jnp.float32),
    )(*sums_list, *cnts_list)


def kernel(vi, atom_mol_batch, N, W, b):
    n_mols = N.shape[0]
    n_atoms = vi.shape[0] - n_mols
    d = W.shape[1]
    chunk_sizes = [n_atoms * f // sum(_FRACS) for f in _FRACS]
    chunk_sizes[-1] += n_atoms - sum(chunk_sizes)
    sums_list, cnts_list = [], []
    row0 = 0
    for cr in chunk_sizes:
        rpw = _GPC * (-(-cr // (_G * _GPC * _NW)))
        n_rows = rpw * _NW
        p_k = _matmul_relu(vi, row0, cr, n_rows * _G, W, b)
        idx_k = lax.dynamic_slice_in_dim(atom_mol_batch, row0, cr)
        idx_pad = jnp.full((n_rows * _G - cr,), n_mols, dtype=jnp.int32)
        idx2d = jnp.concatenate([idx_k, idx_pad]).reshape(n_rows, _G)
        s_k, c_k = _segment_partials_sc(p_k, idx2d)
        sums_list.append(s_k)
        cnts_list.append(c_k.reshape(2, _ACC))
        row0 += cr
    return _combine(sums_list, cnts_list, n_mols, d)


# speedup vs baseline: 1.1974x; 1.0225x over previous
import functools

import jax
import jax.numpy as jnp
from jax import lax
from jax.experimental import pallas as pl
from jax.experimental.pallas import tpu as pltpu
from jax.experimental.pallas import tpu_sc as plsc

_BM = 8000
_FRACS = (3, 2)
_K = len(_FRACS)
_G = 128
_GPC = 8
_NSUB = 16
_NW = 2 * _NSUB
_ACC = 10240


def _mm_body(x_ref, w_ref, b_ref, o_ref):
    y = jnp.dot(x_ref[...], w_ref[...], preferred_element_type=jnp.float32)
    o_ref[...] = jnp.maximum(y + b_ref[...], 0.0)


def _matmul_relu(x, row0, rows, rows_padded, w, b):
    d_in = x.shape[1]
    d_out = w.shape[1]
    off = row0 // _BM
    return pl.pallas_call(
        _mm_body,
        grid=(rows // _BM,),
        in_specs=[
            pl.BlockSpec((_BM, d_in), lambda k: (k + off, 0)),
            pl.BlockSpec((d_in, d_out), lambda k: (0, 0)),
            pl.BlockSpec((1, d_out), lambda k: (0, 0)),
        ],
        out_specs=pl.BlockSpec((_BM, d_out), lambda k: (k, 0)),
        out_shape=jax.ShapeDtypeStruct((rows_padded, d_out), jnp.float32),
    )(x, w, b.reshape(1, d_out))


def _segment_partials_sc(p, idx2d):
    d = p.shape[1]
    n_rows = idx2d.shape[0]
    rpw = n_rows // _NW
    n_groups = rpw // _GPC
    mpt = _ACC // _NSUB
    assert rpw * _NW == n_rows and rpw % _GPC == 0
    mesh = plsc.VectorSubcoreMesh(core_axis_name="c", subcore_axis_name="s")

    @functools.partial(
        pl.kernel,
        mesh=mesh,
        out_type=(
            jax.ShapeDtypeStruct((2, _ACC, d), jnp.float32),
            jax.ShapeDtypeStruct((2, 1, _ACC), jnp.float32),
        ),
        scratch_types=[
            pltpu.VMEM((_G, d), jnp.float32),
            pltpu.VMEM((_G, d), jnp.float32),
            pltpu.VMEM((n_groups, _GPC, _G), jnp.int32),
            pltpu.VMEM((_G,), jnp.float32),
            pltpu.VMEM((mpt,), jnp.float32),
            pltpu.SemaphoreType.DMA,
            pltpu.SemaphoreType.DMA,
            pltpu.SemaphoreType.DMA,
            pltpu.VMEM_SHARED((_ACC, d), jnp.float32),
            pltpu.VMEM_SHARED((_ACC,), jnp.float32),
        ],
    )
    def seg_part(p_hbm, idx_hbm, sums_out, cnts_out, buf0, buf1, idx_v,
                 ones_v, zero_v, sem0, sem1, csem, sums_sh, cnts_sh):
        c = lax.axis_index("c")
        s = lax.axis_index("s")
        w = c * _NSUB + s
        bufs = (buf0, buf1)
        sems = (sem0, sem1)

        zeros16 = jnp.zeros((16,), jnp.float32)
        ones16 = jnp.ones((16,), jnp.float32)

        def zero_body(m, carry):
            for j in range(d // 16):
                buf0[m, pl.ds(16 * j, 16)] = zeros16
            return carry

        lax.fori_loop(0, _G, zero_body, 0)
        for k in range(_G // 16):
            ones_v[pl.ds(16 * k, 16)] = ones16
        for k in range(mpt // 16):
            zero_v[pl.ds(16 * k, 16)] = zeros16

        z0 = s * mpt
        for k in range(mpt // _G):
            pltpu.sync_copy(buf0, sums_sh.at[pl.ds(z0 + k * _G, _G)])
        pltpu.sync_copy(zero_v, cnts_sh.at[pl.ds(z0, mpt)])
        plsc.subcore_barrier()

        row_base = w * rpw

        def src(pc):
            return p_hbm.at[pl.ds((row_base + pc) * _G, _G)]

        for pc in range(2):
            pltpu.async_copy(src(pc), bufs[pc], sems[pc])

        def group_body(t, carry):
            base = t * _GPC
            pltpu.sync_copy(idx_hbm.at[pl.ds(row_base + base, _GPC)],
                            idx_v.at[t])
            for j in range(_GPC):
                b = j % 2
                pc = base + j
                pltpu.make_async_copy(src(pc), bufs[b], sems[b]).wait()
                pltpu.sync_copy(bufs[b], sums_sh.at[idx_v.at[t].at[j]],
                                add=True)
                pltpu.async_copy(ones_v, cnts_sh.at[idx_v.at[t].at[j]],
                                 csem, add=True)

                @pl.when(pc + 2 < rpw)
                def _next():
                    pltpu.async_copy(src(pc + 2), bufs[b], sems[b])
            return carry

        lax.fori_loop(0, n_groups, group_body, 0)

        for t in range(n_groups):
            for j in range(_GPC):
                pltpu.make_async_copy(
                    ones_v, cnts_sh.at[idx_v.at[t].at[j]], csem).wait()
        plsc.subcore_barrier()

        pltpu.sync_copy(sums_sh.at[pl.ds(z0, mpt)],
                        sums_out.at[c].at[pl.ds(z0, mpt)])

        @pl.when(s == 0)
        def _dump_counts():
            pltpu.sync_copy(cnts_sh, cnts_out.at[c].at[0])

    return seg_part(p, idx2d)


_CM = 1024


def _combine_body(*refs):
    s_refs = refs[:_K]
    c_refs = refs[_K:2 * _K]
    o_ref = refs[2 * _K]
    cs = sum(c_ref[0] + c_ref[1] for c_ref in c_refs)
    ss = sum(s_ref[0] + s_ref[1] for s_ref in s_refs)
    r = 1.0 / jnp.maximum(cs, 1.0)
    o_ref[...] = ss * r[:, None]


def _combine(sums_list, cnts_list, n_mols, d):
    grid = (-(-n_mols // _CM),)
    return pl.pallas_call(
        _combine_body,
        grid=grid,
        in_specs=(
            [pl.BlockSpec((2, _CM, d), lambda k: (0, k, 0))] * _K
            + [pl.BlockSpec((2, _CM), lambda k: (0, k))] * _K
        ),
        out_specs=pl.BlockSpec((_CM, d), lambda k: (k, 0)),
        out_shape=jax.ShapeDtypeStruct((n_mols, d), jnp.float32),
    )(*sums_list, *cnts_list)


def kernel(vi, atom_mol_batch, N, W, b):
    n_mols = N.shape[0]
    n_atoms = vi.shape[0] - n_mols
    d = W.shape[1]
    chunk_sizes = [n_atoms * f // sum(_FRACS) for f in _FRACS]
    chunk_sizes[-1] += n_atoms - sum(chunk_sizes)
    sums_list, cnts_list = [], []
    row0 = 0
    for cr in chunk_sizes:
        rpw = _GPC * (-(-cr // (_G * _GPC * _NW)))
        n_rows = rpw * _NW
        p_k = _matmul_relu(vi, row0, cr, n_rows * _G, W, b)
        idx_k = lax.dynamic_slice_in_dim(atom_mol_batch, row0, cr)
        idx_pad = jnp.full((n_rows * _G - cr,), n_mols, dtype=jnp.int32)
        idx2d = jnp.concatenate([idx_k, idx_pad]).reshape(n_rows, _G)
        s_k, c_k = _segment_partials_sc(p_k, idx2d)
        sums_list.append(s_k)
        cnts_list.append(c_k.reshape(2, _ACC))
        row0 += cr
    return _combine(sums_list, cnts_list, n_mols, d)


# speedup vs baseline: 1.2020x; 1.0039x over previous
import functools

import jax
import jax.numpy as jnp
from jax import lax
from jax.experimental import pallas as pl
from jax.experimental.pallas import tpu as pltpu
from jax.experimental.pallas import tpu_sc as plsc

_BM = 16000
_FRACS = (3, 2)
_K = len(_FRACS)
_G = 128
_GPC = 8
_NSUB = 16
_NW = 2 * _NSUB
_ACC = 10240


def _mm_body(x_ref, w_ref, b_ref, o_ref):
    y = jnp.dot(x_ref[...], w_ref[...], preferred_element_type=jnp.float32)
    o_ref[...] = jnp.maximum(y + b_ref[...], 0.0)


def _matmul_relu(x, row0, rows, rows_padded, w, b):
    d_in = x.shape[1]
    d_out = w.shape[1]
    off = row0 // _BM
    return pl.pallas_call(
        _mm_body,
        grid=(rows // _BM,),
        in_specs=[
            pl.BlockSpec((_BM, d_in), lambda k: (k + off, 0)),
            pl.BlockSpec((d_in, d_out), lambda k: (0, 0)),
            pl.BlockSpec((1, d_out), lambda k: (0, 0)),
        ],
        out_specs=pl.BlockSpec((_BM, d_out), lambda k: (k, 0)),
        out_shape=jax.ShapeDtypeStruct((rows_padded, d_out), jnp.float32),
    )(x, w, b.reshape(1, d_out))


def _segment_partials_sc(p, idx2d):
    d = p.shape[1]
    n_rows = idx2d.shape[0]
    rpw = n_rows // _NW
    n_groups = rpw // _GPC
    mpt = _ACC // _NSUB
    assert rpw * _NW == n_rows and rpw % _GPC == 0
    mesh = plsc.VectorSubcoreMesh(core_axis_name="c", subcore_axis_name="s")

    @functools.partial(
        pl.kernel,
        mesh=mesh,
        out_type=(
            jax.ShapeDtypeStruct((2, _ACC, d), jnp.float32),
            jax.ShapeDtypeStruct((2, 1, _ACC), jnp.float32),
        ),
        scratch_types=[
            pltpu.VMEM((_G, d), jnp.float32),
            pltpu.VMEM((_G, d), jnp.float32),
            pltpu.VMEM((n_groups, _GPC, _G), jnp.int32),
            pltpu.VMEM((_G,), jnp.float32),
            pltpu.VMEM((mpt,), jnp.float32),
            pltpu.SemaphoreType.DMA,
            pltpu.SemaphoreType.DMA,
            pltpu.SemaphoreType.DMA,
            pltpu.VMEM_SHARED((_ACC, d), jnp.float32),
            pltpu.VMEM_SHARED((_ACC,), jnp.float32),
        ],
    )
    def seg_part(p_hbm, idx_hbm, sums_out, cnts_out, buf0, buf1, idx_v,
                 ones_v, zero_v, sem0, sem1, csem, sums_sh, cnts_sh):
        c = lax.axis_index("c")
        s = lax.axis_index("s")
        w = c * _NSUB + s
        bufs = (buf0, buf1)
        sems = (sem0, sem1)

        zeros16 = jnp.zeros((16,), jnp.float32)
        ones16 = jnp.ones((16,), jnp.float32)

        def zero_body(m, carry):
            for j in range(d // 16):
                buf0[m, pl.ds(16 * j, 16)] = zeros16
            return carry

        lax.fori_loop(0, _G, zero_body, 0)
        for k in range(_G // 16):
            ones_v[pl.ds(16 * k, 16)] = ones16
        for k in range(mpt // 16):
            zero_v[pl.ds(16 * k, 16)] = zeros16

        z0 = s * mpt
        for k in range(mpt // _G):
            pltpu.sync_copy(buf0, sums_sh.at[pl.ds(z0 + k * _G, _G)])
        pltpu.sync_copy(zero_v, cnts_sh.at[pl.ds(z0, mpt)])
        plsc.subcore_barrier()

        row_base = w * rpw

        def src(pc):
            return p_hbm.at[pl.ds((row_base + pc) * _G, _G)]

        for pc in range(2):
            pltpu.async_copy(src(pc), bufs[pc], sems[pc])

        def group_body(t, carry):
            base = t * _GPC
            pltpu.sync_copy(idx_hbm.at[pl.ds(row_base + base, _GPC)],
                            idx_v.at[t])
            for j in range(_GPC):
                b = j % 2
                pc = base + j
                pltpu.make_async_copy(src(pc), bufs[b], sems[b]).wait()
                pltpu.sync_copy(bufs[b], sums_sh.at[idx_v.at[t].at[j]],
                                add=True)
                pltpu.async_copy(ones_v, cnts_sh.at[idx_v.at[t].at[j]],
                                 csem, add=True)

                @pl.when(pc + 2 < rpw)
                def _next():
                    pltpu.async_copy(src(pc + 2), bufs[b], sems[b])
            return carry

        lax.fori_loop(0, n_groups, group_body, 0)

        for t in range(n_groups):
            for j in range(_GPC):
                pltpu.make_async_copy(
                    ones_v, cnts_sh.at[idx_v.at[t].at[j]], csem).wait()
        plsc.subcore_barrier()

        pltpu.sync_copy(sums_sh.at[pl.ds(z0, mpt)],
                        sums_out.at[c].at[pl.ds(z0, mpt)])

        @pl.when(s == 0)
        def _dump_counts():
            pltpu.sync_copy(cnts_sh, cnts_out.at[c].at[0])

    return seg_part(p, idx2d)


_CM = 1024


def _combine_body(*refs):
    s_refs = refs[:_K]
    c_refs = refs[_K:2 * _K]
    o_ref = refs[2 * _K]
    cs = sum(c_ref[0] + c_ref[1] for c_ref in c_refs)
    ss = sum(s_ref[0] + s_ref[1] for s_ref in s_refs)
    r = 1.0 / jnp.maximum(cs, 1.0)
    o_ref[...] = ss * r[:, None]


def _combine(sums_list, cnts_list, n_mols, d):
    grid = (-(-n_mols // _CM),)
    return pl.pallas_call(
        _combine_body,
        grid=grid,
        in_specs=(
            [pl.BlockSpec((2, _CM, d), lambda k: (0, k, 0))] * _K
            + [pl.BlockSpec((2, _CM), lambda k: (0, k))] * _K
        ),
        out_specs=pl.BlockSpec((_CM, d), lambda k: (k, 0)),
        out_shape=jax.ShapeDtypeStruct((n_mols, d), jnp.float32),
    )(*sums_list, *cnts_list)


def kernel(vi, atom_mol_batch, N, W, b):
    n_mols = N.shape[0]
    n_atoms = vi.shape[0] - n_mols
    d = W.shape[1]
    chunk_sizes = [n_atoms * f // sum(_FRACS) for f in _FRACS]
    chunk_sizes[-1] += n_atoms - sum(chunk_sizes)
    sums_list, cnts_list = [], []
    row0 = 0
    for cr in chunk_sizes:
        rpw = _GPC * (-(-cr // (_G * _GPC * _NW)))
        n_rows = rpw * _NW
        p_k = _matmul_relu(vi, row0, cr, n_rows * _G, W, b)
        idx_k = lax.dynamic_slice_in_dim(atom_mol_batch, row0, cr)
        idx_pad = jnp.full((n_rows * _G - cr,), n_mols, dtype=jnp.int32)
        idx2d = jnp.concatenate([idx_k, idx_pad]).reshape(n_rows, _G)
        s_k, c_k = _segment_partials_sc(p_k, idx2d)
        sums_list.append(s_k)
        cnts_list.append(c_k.reshape(2, _ACC))
        row0 += cr
    return _combine(sums_list, cnts_list, n_mols, d)
